# Initial kernel scaffold; baseline (speedup 1.0000x reference)
import jax
import jax.numpy as jnp
from jax import lax
from jax.experimental import pallas as pl
from jax.experimental.pallas import tpu as pltpu
from jax.experimental.pallas import tpu_sc as plsc

N = 10000
D = 128
E = 320000

NC = 2
NS = 16
L = 16

DH = D // NC
W = DH + L
EPT = E // NS
CHUNK = 125
NCHUNK = EPT // CHUNK
RPT = N // NS
RCH = 125
NRCH = RPT // RCH


def _body(table_hbm, dst2_hbm, src_hbm, out_hbm,
          dst_idx_v, src_idx_v, rows_v, div_v, out_v, acc_s):
    c = lax.axis_index("c")
    s = lax.axis_index("s")
    row0 = s * RPT

    def _zero_row(r, _):
        for k in range(W // L):
            div_v[r, pl.ds(k * L, L)] = jnp.zeros((L,), jnp.float32)
        return _
    lax.fori_loop(0, RCH, _zero_row, None)
    for k in range(NRCH):
        pltpu.sync_copy(div_v, acc_s.at[pl.ds(row0 + k * RCH, RCH)])

    pltpu.sync_copy(dst2_hbm.at[c, s], dst_idx_v)
    pltpu.sync_copy(src_hbm.at[s], src_idx_v)

    plsc.subcore_barrier()

    def _edge_chunk(j, _):
        pltpu.sync_copy(table_hbm.at[dst_idx_v.at[j]], rows_v)
        pltpu.sync_copy(rows_v, acc_s.at[src_idx_v.at[j]], add=True)
        return _
    lax.fori_loop(0, NCHUNK, _edge_chunk, None)

    plsc.subcore_barrier()

    for k in range(NRCH):
        base = row0 + k * RCH
        pltpu.sync_copy(acc_s.at[pl.ds(base, RCH)], div_v)

        def _div_row(r, _):
            deg = div_v[r, pl.ds(DH, L)]
            recip = 1.0 / deg
            for q in range(DH // L):
                out_v[r, pl.ds(q * L, L)] = div_v[r, pl.ds(q * L, L)] * recip
            return _
        lax.fori_loop(0, RCH, _div_row, None)
        pltpu.sync_copy(out_v, out_hbm.at[pl.ds(base, RCH), pl.ds(c * DH, DH)])


def kernel(features, edge_index):
    src = edge_index[0]
    dst = edge_index[1]
    ones = jnp.ones((N, L), jnp.float32)
    table = jnp.concatenate([
        jnp.concatenate([features[:, :DH], ones], axis=1),
        jnp.concatenate([features[:, DH:], ones], axis=1),
    ], axis=0)
    dst2 = jnp.stack([dst, dst + N]).reshape(NC, NS, NCHUNK, CHUNK)
    src_r = src.reshape(NS, NCHUNK, CHUNK)

    mesh = plsc.VectorSubcoreMesh(core_axis_name="c", subcore_axis_name="s")
    k = pl.kernel(
        _body,
        out_type=jax.ShapeDtypeStruct((N, D), jnp.float32),
        mesh=mesh,
        scratch_types=[
            pltpu.VMEM((NCHUNK, CHUNK), jnp.int32),
            pltpu.VMEM((NCHUNK, CHUNK), jnp.int32),
            pltpu.VMEM((CHUNK, W), jnp.float32),
            pltpu.VMEM((RCH, W), jnp.float32),
            pltpu.VMEM((RCH, DH), jnp.float32),
            pltpu.VMEM_SHARED((N, W), jnp.float32),
        ],
    )
    return k(table, dst2, src_r)


# speedup vs baseline: 6.8620x; 6.8620x over previous
import jax
import jax.numpy as jnp
from jax import lax
from jax.experimental import pallas as pl
from jax.experimental.pallas import tpu as pltpu
from jax.experimental.pallas import tpu_sc as plsc

N = 10000
D = 128
E = 320000

NC = 2
NS = 16
L = 16

DH = D // NC
W = DH + L
EPT = E // NS
CHUNK = 125
NCHUNK = EPT // CHUNK
RPT = N // NS
RCH = 125
NRCH = RPT // RCH


def _body(table_hbm, dst2_hbm, src_hbm, out_hbm,
          dst_idx_v, src_idx_v, rows_v, div_v, out_v, acc_s):
    c = lax.axis_index("c")
    s = lax.axis_index("s")
    row0 = s * RPT

    def _zero_row(r, _):
        for k in range(W // L):
            div_v[r, pl.ds(k * L, L)] = jnp.zeros((L,), jnp.float32)
        return _
    lax.fori_loop(0, RCH, _zero_row, None)
    for k in range(NRCH):
        pltpu.sync_copy(div_v, acc_s.at[pl.ds(row0 + k * RCH, RCH)])

    pltpu.sync_copy(dst2_hbm.at[c, s], dst_idx_v)
    pltpu.sync_copy(src_hbm.at[s], src_idx_v)

    plsc.subcore_barrier()

    def _edge_chunk(j, _):
        pltpu.sync_copy(table_hbm.at[dst_idx_v.at[j]], rows_v)
        pltpu.sync_copy(rows_v, acc_s.at[src_idx_v.at[j]], add=True)
        return _
    lax.fori_loop(0, NCHUNK, _edge_chunk, None)

    plsc.subcore_barrier()

    for k in range(NRCH):
        base = row0 + k * RCH
        pltpu.sync_copy(acc_s.at[pl.ds(base, RCH)], div_v)

        def _div_row(r, _):
            deg = div_v[r, pl.ds(DH, L)]
            recip = 1.0 / deg
            for q in range(DH // L):
                out_v[r, pl.ds(q * L, L)] = div_v[r, pl.ds(q * L, L)] * recip
            return _
        lax.fori_loop(0, RCH, _div_row, None)
        pltpu.sync_copy(out_v, out_hbm.at[pl.ds(base, RCH), pl.ds(c * DH, DH)])


def kernel(features, edge_index):
    src = edge_index[0]
    dst = edge_index[1]
    ones = jnp.ones((N, L), jnp.float32)
    table = jnp.concatenate([
        jnp.concatenate([features[:, :DH], ones], axis=1),
        jnp.concatenate([features[:, DH:], ones], axis=1),
    ], axis=0)
    dst2 = jnp.stack([dst, dst + N]).reshape(NC, NS, NCHUNK, CHUNK)
    src_r = src.reshape(NS, NCHUNK, CHUNK)

    mesh = plsc.VectorSubcoreMesh(core_axis_name="c", subcore_axis_name="s")
    k = pl.kernel(
        _body,
        out_type=jax.ShapeDtypeStruct((N, D), jnp.float32),
        mesh=mesh,
        scratch_types=[
            pltpu.VMEM((NCHUNK, CHUNK), jnp.int32),
            pltpu.VMEM((NCHUNK, CHUNK), jnp.int32),
            pltpu.VMEM((CHUNK, W), jnp.float32),
            pltpu.VMEM((RCH, W), jnp.float32),
            pltpu.VMEM((RCH, DH), jnp.float32),
            pltpu.VMEM_SHARED((N, W), jnp.float32),
        ],
        compiler_params=pltpu.CompilerParams(use_tc_tiling_on_sc=False),
    )
    return k(table, dst2, src_r)


# speedup vs baseline: 8.5069x; 1.2397x over previous
import jax
import jax.numpy as jnp
from jax import lax
from jax.experimental import pallas as pl
from jax.experimental.pallas import tpu as pltpu
from jax.experimental.pallas import tpu_sc as plsc

N = 10000
D = 128
E = 320000

NC = 2
NS = 16
L = 16

DH = D // NC
W = DH + L
EPT = E // NS
CHUNK = 125
NCHUNK = EPT // CHUNK
RPT = N // NS
RCH = 125
NRCH = RPT // RCH


NBUF = 2
AHEAD = 1


def _body(table_hbm, dst2_hbm, src_hbm, out_hbm,
          dst_idx_v, src_idx_v, rows_v, div_v, out_v, acc_s, gsem, ssem):
    c = lax.axis_index("c")
    s = lax.axis_index("s")
    row0 = s * RPT

    def fire_gather(chunk, b):
        pltpu.async_copy(table_hbm.at[dst_idx_v.at[chunk]], rows_v.at[b],
                         gsem.at[b])

    def wait_gather(b):
        pltpu.make_async_copy(table_hbm.at[dst_idx_v.at[0]], rows_v.at[b],
                              gsem.at[b]).wait()

    def fire_scatter(chunk, b):
        pltpu.async_copy(rows_v.at[b], acc_s.at[src_idx_v.at[chunk]],
                         ssem.at[b], add=True)

    def wait_scatter(b):
        pltpu.make_async_copy(rows_v.at[b], acc_s.at[src_idx_v.at[0]],
                              ssem.at[b]).wait()

    def _zero_row(r, _):
        for k in range(W // L):
            div_v[r, pl.ds(k * L, L)] = jnp.zeros((L,), jnp.float32)
        return _
    lax.fori_loop(0, RCH, _zero_row, None)
    for k in range(NRCH):
        pltpu.sync_copy(div_v, acc_s.at[pl.ds(row0 + k * RCH, RCH)])

    pltpu.sync_copy(dst2_hbm.at[c, s], dst_idx_v)
    pltpu.sync_copy(src_hbm.at[s], src_idx_v)

    plsc.subcore_barrier()

    for ch in range(AHEAD):
        fire_gather(ch, ch % NBUF)
    for ch in range(NBUF):
        wait_gather(ch)
        fire_scatter(ch, ch)
        if ch + AHEAD >= NBUF:
            wait_scatter((ch + AHEAD) % NBUF)
        fire_gather(ch + AHEAD, (ch + AHEAD) % NBUF)

    def _edge_chunk(i, _):
        j = (i + 1) * NBUF
        for b in range(NBUF):
            ch = j + b
            wait_gather(b)
            fire_scatter(ch, b)
            nxt = ch + AHEAD

            @pl.when(nxt < NCHUNK)
            def _():
                b2 = (b + AHEAD) % NBUF
                wait_scatter(b2)
                fire_gather(nxt, b2)
        return _
    lax.fori_loop(0, NCHUNK // NBUF - 1, _edge_chunk, None)

    for b in range(NBUF):
        wait_scatter(b)

    plsc.subcore_barrier()

    for k in range(NRCH):
        base = row0 + k * RCH
        pltpu.sync_copy(acc_s.at[pl.ds(base, RCH)], div_v)

        def _div_row(r, _):
            deg = div_v[r, pl.ds(DH, L)]
            recip = 1.0 / deg
            for q in range(DH // L):
                out_v[r, pl.ds(q * L, L)] = div_v[r, pl.ds(q * L, L)] * recip
            return _
        lax.fori_loop(0, RCH, _div_row, None)
        pltpu.sync_copy(out_v, out_hbm.at[pl.ds(base, RCH), pl.ds(c * DH, DH)])


def kernel(features, edge_index):
    src = edge_index[0]
    dst = edge_index[1]
    ones = jnp.ones((N, L), jnp.float32)
    table = jnp.concatenate([
        jnp.concatenate([features[:, :DH], ones], axis=1),
        jnp.concatenate([features[:, DH:], ones], axis=1),
    ], axis=0)
    dst2 = jnp.stack([dst, dst + N]).reshape(NC, NS, NCHUNK, CHUNK)
    src_r = src.reshape(NS, NCHUNK, CHUNK)

    mesh = plsc.VectorSubcoreMesh(core_axis_name="c", subcore_axis_name="s")
    k = pl.kernel(
        _body,
        out_type=jax.ShapeDtypeStruct((N, D), jnp.float32),
        mesh=mesh,
        scratch_types=[
            pltpu.VMEM((NCHUNK, CHUNK), jnp.int32),
            pltpu.VMEM((NCHUNK, CHUNK), jnp.int32),
            pltpu.VMEM((NBUF, CHUNK, W), jnp.float32),
            pltpu.VMEM((RCH, W), jnp.float32),
            pltpu.VMEM((RCH, DH), jnp.float32),
            pltpu.VMEM_SHARED((N, W), jnp.float32),
            pltpu.SemaphoreType.DMA((NBUF,)),
            pltpu.SemaphoreType.DMA((NBUF,)),
        ],
        compiler_params=pltpu.CompilerParams(use_tc_tiling_on_sc=False),
    )
    return k(table, dst2, src_r)


# speedup vs baseline: 10.5908x; 1.2450x over previous
import jax
import jax.numpy as jnp
from jax import lax
from jax.experimental import pallas as pl
from jax.experimental.pallas import tpu as pltpu
from jax.experimental.pallas import tpu_sc as plsc

N = 10000
D = 128
E = 320000

NC = 2
NS = 16
L = 16

DH = D // NC
W = DH + L
EPT = E // NS
CHUNK = 125
NCHUNK = EPT // CHUNK
RPT = N // NS
RCH = 125
NRCH = RPT // RCH

NBUF = 4
AHEAD = 2
G = 4
NGRP = NCHUNK // G
NSETS = 4


def _body(table_hbm, dst2_hbm, src_hbm, out_hbm,
          dst_idx_v, src_idx_v, rows_v, div_v, out_v, acc_s,
          gsem, ssem, isem):
    c = lax.axis_index("c")
    s = lax.axis_index("s")
    row0 = s * RPT

    def iset(chunk):
        return (chunk // G) % NSETS

    def fire_idx(grp):
        st = grp % NSETS
        pltpu.async_copy(dst2_hbm.at[c, s, grp], dst_idx_v.at[st], isem.at[st])
        pltpu.async_copy(src_hbm.at[s, grp], src_idx_v.at[st], isem.at[st])

    def wait_idx(grp):
        st = grp % NSETS
        pltpu.make_async_copy(dst2_hbm.at[c, s, 0], dst_idx_v.at[st],
                              isem.at[st]).wait()
        pltpu.make_async_copy(src_hbm.at[s, 0], src_idx_v.at[st],
                              isem.at[st]).wait()

    def fire_gather(ch, row, b):
        pltpu.async_copy(table_hbm.at[dst_idx_v.at[iset(ch), row]],
                         rows_v.at[b], gsem.at[b])

    def wait_gather(b):
        pltpu.make_async_copy(table_hbm.at[dst_idx_v.at[0, 0]], rows_v.at[b],
                              gsem.at[b]).wait()

    def fire_scatter(ch, row, b):
        pltpu.async_copy(rows_v.at[b], acc_s.at[src_idx_v.at[iset(ch), row]],
                         ssem.at[b], add=True)

    def wait_scatter(b):
        pltpu.make_async_copy(rows_v.at[b], acc_s.at[src_idx_v.at[0, 0]],
                              ssem.at[b]).wait()

    def _zero_row(r, _):
        for k in range(W // L):
            div_v[r, pl.ds(k * L, L)] = jnp.zeros((L,), jnp.float32)
        return _
    lax.fori_loop(0, RCH, _zero_row, None)
    for k in range(NRCH):
        pltpu.sync_copy(div_v, acc_s.at[pl.ds(row0 + k * RCH, RCH)])

    for grp in range(min(3, NGRP)):
        fire_idx(grp)
    wait_idx(0)

    plsc.subcore_barrier()

    for ch in range(AHEAD):
        fire_gather(ch, ch % G, ch % NBUF)
    for ch in range(NBUF):
        if ch == AHEAD:
            wait_idx(1)
        wait_gather(ch)
        fire_scatter(ch, ch % G, ch)
        if ch + AHEAD >= NBUF:
            wait_scatter((ch + AHEAD) % NBUF)
        fire_gather(ch + AHEAD, (ch + AHEAD) % G, (ch + AHEAD) % NBUF)

    def _group(i, _):
        g = i + 1
        j = g * G

        @pl.when(g + 2 < NGRP)
        def _():
            fire_idx(g + 2)

        for b in range(NBUF):
            ch = j + b
            if b == AHEAD:
                @pl.when(g + 1 < NGRP)
                def _():
                    wait_idx(g + 1)
            wait_gather(b)
            fire_scatter(ch, b, b)
            nxt = ch + AHEAD

            @pl.when(nxt < NCHUNK)
            def _():
                b2 = (b + AHEAD) % NBUF
                wait_scatter(b2)
                fire_gather(nxt, (b + AHEAD) % G, b2)
        return _
    lax.fori_loop(0, NGRP - 1, _group, None)

    for b in range(NBUF):
        wait_scatter(b)

    plsc.subcore_barrier()

    for k in range(NRCH):
        base = row0 + k * RCH
        pltpu.sync_copy(acc_s.at[pl.ds(base, RCH)], div_v)

        def _div_row(r, _):
            deg = div_v[r, pl.ds(DH, L)]
            recip = 1.0 / deg
            for q in range(DH // L):
                out_v[r, pl.ds(q * L, L)] = div_v[r, pl.ds(q * L, L)] * recip
            return _
        lax.fori_loop(0, RCH, _div_row, None)
        pltpu.sync_copy(out_v, out_hbm.at[pl.ds(base, RCH), pl.ds(c * DH, DH)])


def kernel(features, edge_index):
    src = edge_index[0]
    dst = edge_index[1]
    ones = jnp.ones((N, L), jnp.float32)
    table = jnp.concatenate([
        jnp.concatenate([features[:, :DH], ones], axis=1),
        jnp.concatenate([features[:, DH:], ones], axis=1),
    ], axis=0)
    dst2 = jnp.stack([dst, dst + N]).reshape(NC, NS, NGRP, G, CHUNK)
    src_r = src.reshape(NS, NGRP, G, CHUNK)

    mesh = plsc.VectorSubcoreMesh(core_axis_name="c", subcore_axis_name="s")
    k = pl.kernel(
        _body,
        out_type=jax.ShapeDtypeStruct((N, D), jnp.float32),
        mesh=mesh,
        scratch_types=[
            pltpu.VMEM((NSETS, G, CHUNK), jnp.int32),
            pltpu.VMEM((NSETS, G, CHUNK), jnp.int32),
            pltpu.VMEM((NBUF, CHUNK, W), jnp.float32),
            pltpu.VMEM((RCH, W), jnp.float32),
            pltpu.VMEM((RCH, DH), jnp.float32),
            pltpu.VMEM_SHARED((N, W), jnp.float32),
            pltpu.SemaphoreType.DMA((NBUF,)),
            pltpu.SemaphoreType.DMA((NBUF,)),
            pltpu.SemaphoreType.DMA((NSETS,)),
        ],
        compiler_params=pltpu.CompilerParams(use_tc_tiling_on_sc=False),
    )
    return k(table, dst2, src_r)
